# baseline (speedup 1.0000x reference)
import jax
import jax.numpy as jnp
from jax import lax
from jax.experimental import pallas as pl
from jax.experimental.pallas import tpu as pltpu
from jax.experimental.pallas import tpu_sc as plsc

N = 10000
D = 256
DH = 128
NNZ = 160000

NC = 2
NS = 16
NPAD = 10112
NNZP = 163840
EPT = NNZP // NS
E = 80
NCHUNK = EPT // E
GRP = 4
RPT = NPAD // NS
CH = 40
NCH = N // CH
KMAX = (NCH + NS - 1) // NS
NSLOT = 2 * NPAD
ZSLOT = 7


def _body(xh, rows2, cols2, vals2, zr, out, xs,
          g0, g1, s0, s1, ic0, ic1, iv0, iv1, ir0, ir1, ir2, ir3, acc,
          smg0, smg1, sms0, sms1, smi0, smi1):
    c = lax.axis_index("c")
    s = lax.axis_index("s")
    coff = c * NPAD

    G = [g0, g1]
    S = [s0, s1]
    IC = [ic0, ic1]
    IV = [iv0, iv1]
    IR = [ir0, ir1, ir2, ir3]
    SG = [smg0, smg1]
    SS = [sms0, sms1]
    SI = [smi0, smi1]

    my_rows = pl.ds(coff + s * RPT, RPT)
    pltpu.sync_copy(xh.at[my_rows], xs.at[pl.ds(coff + s * RPT, RPT)])
    pltpu.sync_copy(zr.at[my_rows],
                    xs.at[pl.ds(ZSLOT * NSLOT + coff + s * RPT, RPT)])

    ebase0 = s * EPT

    def issue_idx(p, k, eoff):
        p2, p4 = p % 2, p % 4
        base = eoff + ebase0 + k * E
        pltpu.async_copy(cols2.at[pl.ds(base, E)], IC[p2], SI[p2])
        pltpu.async_copy(vals2.at[pl.ds(base, E)], IV[p2], SI[p2])
        pltpu.async_copy(rows2.at[pl.ds(base, E)], IR[p4], SI[p2])

    def wait_idx(p, k, eoff):
        p2, p4 = p % 2, p % 4
        base = eoff + ebase0 + k * E
        pltpu.make_async_copy(cols2.at[pl.ds(base, E)], IC[p2], SI[p2]).wait()
        pltpu.make_async_copy(vals2.at[pl.ds(base, E)], IV[p2], SI[p2]).wait()
        pltpu.make_async_copy(rows2.at[pl.ds(base, E)], IR[p4], SI[p2]).wait()

    def shift_cols(p, goff):
        p2 = p % 2
        for i in range(E // 16):
            sl = pl.ds(i * 16, 16)
            IC[p2][sl] = IC[p2][sl] + goff

    def issue_gather(p):
        p2 = p % 2
        pltpu.async_copy(xs.at[IC[p2]], G[p2], SG[p2])

    def wait_gather(p):
        p2 = p % 2
        pltpu.make_async_copy(xs.at[IC[p2]], G[p2], SG[p2]).wait()

    def scale(p):
        p2 = p % 2
        gb, sb, vb = G[p2], S[p2], IV[p2]

        @plsc.parallel_loop(0, E // 16)
        def _(g):
            vv = vb[pl.ds(g * 16, 16)]
            for t in range(16):
                v = vv[t]
                e = g * 16 + t
                for j in range(DH // 16):
                    sl = pl.ds(j * 16, 16)
                    sb[e, sl] = gb[e, sl] * v

    def issue_scatter(p):
        p2, p4 = p % 2, p % 4
        pltpu.async_copy(S[p2], acc.at[IR[p4]], SS[p2], add=True)

    def wait_scatter(p):
        p2, p4 = p % 2, p % 4
        pltpu.make_async_copy(S[p2], acc.at[IR[p4]], SS[p2]).wait()

    def spmm_body(i, carry):
        parity = lax.rem(i, 2)
        is_even = parity == 0
        eoff = parity * NNZP
        goff = i * NSLOT + coff
        init_slot = jnp.where(is_even, ZSLOT, i - 1)

        pltpu.sync_copy(
            xs.at[pl.ds(init_slot * NSLOT + coff + s * RPT, RPT)],
            acc.at[pl.ds(s * RPT, RPT)])
        plsc.subcore_barrier()

        issue_idx(0, 0, eoff)
        issue_idx(1, 1, eoff)
        wait_idx(0, 0, eoff)
        shift_cols(0, goff)
        issue_gather(0)

        def group_body(g, cy):
            for p in range(GRP):
                k = g * GRP + p
                wait_gather(p)

                @pl.when(k < NCHUNK - 1)
                def _():
                    wait_idx(p + 1, k + 1, eoff)
                    shift_cols(p + 1, goff)
                    issue_gather(p + 1)

                @pl.when(k >= 2)
                def _():
                    wait_scatter(p + 2)

                scale(p)
                issue_scatter(p)

                @pl.when(k < NCHUNK - 2)
                def _():
                    issue_idx(p + 2, k + 2, eoff)

            return cy

        lax.fori_loop(0, NCHUNK // GRP, group_body, 0)

        wait_scatter(NCHUNK - 2)
        wait_scatter(NCHUNK - 1)
        plsc.subcore_barrier()
        pltpu.sync_copy(acc.at[pl.ds(s * RPT, RPT)],
                        xs.at[pl.ds((i + 1) * NSLOT + coff + s * RPT, RPT)])
        plsc.subcore_barrier()
        return carry

    lax.fori_loop(0, 6, spmm_body, 0)

    B4 = [g0, g1, s0, s1]
    for k in range(KMAX):
        cid = s + k * NS

        @pl.when(cid < NCH)
        def _():
            r0 = cid * CH
            for t in range(4):
                pltpu.sync_copy(
                    xs.at[pl.ds(2 * t * NSLOT + coff + r0, CH)],
                    B4[t].at[pl.ds(0, CH)])

            def mean_body(r, cy):
                for j in range(DH // 16):
                    sl = pl.ds(j * 16, 16)
                    g0[r, sl] = (g0[r, sl] + g1[r, sl] + s0[r, sl]
                                 + s1[r, sl]) * 0.25
                return cy

            lax.fori_loop(0, CH, mean_body, 0)
            pltpu.sync_copy(g0.at[pl.ds(0, CH)],
                            out.at[pl.ds(r0, CH), pl.ds(c * DH, DH)])


_mesh = plsc.VectorSubcoreMesh(core_axis_name="c", subcore_axis_name="s")

_call = pl.kernel(
    _body,
    out_type=jax.ShapeDtypeStruct((N, D), jnp.float32),
    mesh=_mesh,
    scratch_types=[
        pltpu.HBM((8 * NSLOT, DH), jnp.float32),
        pltpu.VMEM((E, DH), jnp.float32),
        pltpu.VMEM((E, DH), jnp.float32),
        pltpu.VMEM((E, DH), jnp.float32),
        pltpu.VMEM((E, DH), jnp.float32),
        pltpu.VMEM((E,), jnp.int32),
        pltpu.VMEM((E,), jnp.int32),
        pltpu.VMEM((E,), jnp.float32),
        pltpu.VMEM((E,), jnp.float32),
        pltpu.VMEM((E,), jnp.int32),
        pltpu.VMEM((E,), jnp.int32),
        pltpu.VMEM((E,), jnp.int32),
        pltpu.VMEM((E,), jnp.int32),
        pltpu.VMEM_SHARED((NPAD, DH), jnp.float32),
    ] + [pltpu.SemaphoreType.DMA] * 6,
)


def kernel(pois_embs, src_indices, src_values, tar_indices, tar_values):
    xh = jnp.zeros((2 * NPAD, DH), jnp.float32)
    xh = xh.at[:N].set(pois_embs[:, :DH]).at[NPAD:NPAD + N].set(pois_embs[:, DH:])

    def pad_edges(indices, values):
        rows = jnp.zeros((NNZP,), jnp.int32).at[:NNZ].set(
            indices[0].astype(jnp.int32))
        cols = jnp.zeros((NNZP,), jnp.int32).at[:NNZ].set(
            indices[1].astype(jnp.int32))
        vals = jnp.zeros((NNZP,), jnp.float32).at[:NNZ].set(values)
        return rows, cols, vals

    tr, tcl, tv = pad_edges(tar_indices, tar_values)
    sr, scl, sv = pad_edges(src_indices, src_values)
    tcl, tr, tv = lax.sort((tcl, tr, tv), num_keys=1)
    scl, sr, sv = lax.sort((scl, sr, sv), num_keys=1)
    rows2 = jnp.concatenate([tr, sr])
    cols2 = jnp.concatenate([tcl, scl])
    vals2 = jnp.concatenate([tv, sv])
    zr = jnp.zeros((2 * NPAD, DH), jnp.float32)
    return _call(xh, rows2, cols2, vals2, zr)


# speedup vs baseline: 1.2833x; 1.2833x over previous
import jax
import jax.numpy as jnp
from jax import lax
from jax.experimental import pallas as pl
from jax.experimental.pallas import tpu as pltpu
from jax.experimental.pallas import tpu_sc as plsc

N = 10000
D = 256
DH = 128
NNZ = 160000

NC = 2
NS = 16
NPAD = 10112
NNZP = 163840
EPT = NNZP // NS
E = 320
NCHUNK = EPT // E
RPT = NPAD // NS
CH = 40
NCH = N // CH
KMAX = (NCH + NS - 1) // NS
CSTRIDE = NNZP + E
SLOT = 2 * NPAD
MSLOT = 4
ZSLOT = 5


def _body(xh, tr, tcl, tv, sr, scl, sv, zr, out, xs,
          ic0, ic1, iv0, iv1, ir0, ir1, rows_buf, acc,
          gsem, isem0, isem1):
    c = lax.axis_index("c")
    s = lax.axis_index("s")
    coff = c * NPAD

    IC = [ic0, ic1]
    IV = [iv0, iv1]
    IR = [ir0, ir1]
    SI = [isem0, isem1]

    my_rows = pl.ds(coff + s * RPT, RPT)
    pltpu.sync_copy(xh.at[my_rows], xs.at[pl.ds(coff + s * RPT, RPT)])
    pltpu.sync_copy(zr.at[my_rows],
                    xs.at[pl.ds(ZSLOT * SLOT + coff + s * RPT, RPT)])

    base0 = s * EPT
    cbase0 = c * CSTRIDE + base0

    def spmm(rows_hbm, cols_hbm, vals_hbm, in_slot, init_slot, dst_slot):
        soff = in_slot * SLOT
        pltpu.sync_copy(
            xs.at[pl.ds(init_slot * SLOT + coff + s * RPT, RPT)],
            acc.at[pl.ds(s * RPT, RPT)])
        plsc.subcore_barrier()

        def issue_idx(k, p):
            base = base0 + k * E
            pltpu.async_copy(cols_hbm.at[pl.ds(cbase0 + k * E, E)], IC[p], SI[p])
            pltpu.async_copy(vals_hbm.at[pl.ds(base, E)], IV[p], SI[p])
            pltpu.async_copy(rows_hbm.at[pl.ds(base, E)], IR[p], SI[p])

        def wait_idx(k, p):
            base = base0 + k * E
            pltpu.make_async_copy(cols_hbm.at[pl.ds(cbase0 + k * E, E)], IC[p],
                                  SI[p]).wait()
            pltpu.make_async_copy(vals_hbm.at[pl.ds(base, E)], IV[p],
                                  SI[p]).wait()
            pltpu.make_async_copy(rows_hbm.at[pl.ds(base, E)], IR[p],
                                  SI[p]).wait()

        def step(k, p):
            wait_idx(k, p)
            issue_idx(k + 1, 1 - p)

            cb = IC[p]

            @plsc.parallel_loop(0, E // 16)
            def _(g):
                sl = pl.ds(g * 16, 16)
                cb[sl] = cb[sl] + soff

            pltpu.async_copy(xs.at[cb], rows_buf, gsem).wait()

            vb = IV[p]

            @plsc.parallel_loop(0, E // 16)
            def _(g):
                vv = vb[pl.ds(g * 16, 16)]
                for t in range(16):
                    v = vv[t]
                    e = g * 16 + t
                    for j in range(DH // 16):
                        sl = pl.ds(j * 16, 16)
                        rows_buf[e, sl] = rows_buf[e, sl] * v

            pltpu.sync_copy(rows_buf, acc.at[IR[p]], add=True)

        issue_idx(0, 0)

        def pair_body(i, carry):
            k0 = 2 * i
            step(k0, 0)
            step(k0 + 1, 1)
            return carry

        lax.fori_loop(0, NCHUNK // 2, pair_body, 0)
        wait_idx(NCHUNK, 0)
        plsc.subcore_barrier()
        pltpu.sync_copy(acc.at[pl.ds(s * RPT, RPT)],
                        xs.at[pl.ds(dst_slot * SLOT + coff + s * RPT, RPT)])
        plsc.subcore_barrier()

    def layer_body(i, carry):
        spmm(tr, tcl, tv, i, ZSLOT, MSLOT)
        spmm(sr, scl, sv, MSLOT, i, i + 1)
        return carry

    lax.fori_loop(0, 3, layer_body, 0)

    def mean_chunk(k, carry):
        cid = s + k * NS

        @pl.when(cid < NCH)
        def _():
            r0 = cid * CH
            for t in range(4):
                pltpu.sync_copy(
                    xs.at[pl.ds(t * SLOT + coff + r0, CH)],
                    rows_buf.at[pl.ds(t * CH, CH)])

            def mean_body(r, cy):
                for j in range(DH // 16):
                    sl = pl.ds(j * 16, 16)
                    rows_buf[r, sl] = (rows_buf[r, sl] + rows_buf[CH + r, sl]
                                       + rows_buf[2 * CH + r, sl]
                                       + rows_buf[3 * CH + r, sl]) * 0.25
                return cy

            lax.fori_loop(0, CH, mean_body, 0)
            pltpu.sync_copy(rows_buf.at[pl.ds(0, CH)],
                            out.at[pl.ds(r0, CH), pl.ds(c * DH, DH)])

        return carry

    lax.fori_loop(0, KMAX, mean_chunk, 0)


_mesh = plsc.VectorSubcoreMesh(core_axis_name="c", subcore_axis_name="s")

_call = pl.kernel(
    _body,
    out_type=jax.ShapeDtypeStruct((N, D), jnp.float32),
    mesh=_mesh,
    scratch_types=[
        pltpu.HBM((6 * SLOT, DH), jnp.float32),
        pltpu.VMEM((E,), jnp.int32),
        pltpu.VMEM((E,), jnp.int32),
        pltpu.VMEM((E,), jnp.float32),
        pltpu.VMEM((E,), jnp.float32),
        pltpu.VMEM((E,), jnp.int32),
        pltpu.VMEM((E,), jnp.int32),
        pltpu.VMEM((E, DH), jnp.float32),
        pltpu.VMEM_SHARED((NPAD, DH), jnp.float32),
        pltpu.SemaphoreType.DMA,
        pltpu.SemaphoreType.DMA,
        pltpu.SemaphoreType.DMA,
    ],
)


def kernel(pois_embs, src_indices, src_values, tar_indices, tar_values):
    xh = jnp.zeros((2 * NPAD, DH), jnp.float32)
    xh = xh.at[:N].set(pois_embs[:, :DH]).at[NPAD:NPAD + N].set(pois_embs[:, DH:])

    def pad_edges(indices, values):
        rows = jnp.zeros((CSTRIDE,), jnp.int32).at[:NNZ].set(
            indices[0].astype(jnp.int32))
        cols = jnp.zeros((CSTRIDE,), jnp.int32).at[:NNZ].set(
            indices[1].astype(jnp.int32))
        vals = jnp.zeros((CSTRIDE,), jnp.float32).at[:NNZ].set(values)
        cols2 = jnp.concatenate([cols, cols + NPAD])
        return rows, cols2, vals

    tr, tcl, tv = pad_edges(tar_indices, tar_values)
    sr, scl, sv = pad_edges(src_indices, src_values)
    zr = jnp.zeros((2 * NPAD, DH), jnp.float32)
    return _call(xh, tr, tcl, tv, sr, scl, sv, zr)


# speedup vs baseline: 1.3954x; 1.0873x over previous
import jax
import jax.numpy as jnp
from jax import lax
from jax.experimental import pallas as pl
from jax.experimental.pallas import tpu as pltpu
from jax.experimental.pallas import tpu_sc as plsc

N = 10000
D = 256
DH = 128
NNZ = 160000

NC = 2
NS = 16
NPAD = 10112
NNZP = 163840
EPT = NNZP // NS
E = 256
NCHUNK = EPT // E
RPT = NPAD // NS
CH = 40
NCH = N // CH
KMAX = (NCH + NS - 1) // NS
CSTRIDE = NNZP + E


def _body(xh, tr, tcl, tv16, sr, scl, sv16, zr, out,
          m, x1, x2, x3,
          ic0, ic1, iv0, iv1, ir0, ir1, rows_buf, acc,
          gsem, isem0, isem1):
    c = lax.axis_index("c")
    s = lax.axis_index("s")
    coff = c * NPAD

    IC = [ic0, ic1]
    IV = [iv0, iv1]
    IR = [ir0, ir1]
    SI = [isem0, isem1]

    base0 = s * EPT
    cbase0 = c * CSTRIDE + base0

    def spmm(rows_hbm, cols_hbm, vals16_hbm, xsrc, init, dst):
        pltpu.sync_copy(init.at[pl.ds(coff + s * RPT, RPT)],
                        acc.at[pl.ds(s * RPT, RPT)])
        plsc.subcore_barrier()

        def issue_idx(k, p):
            base = base0 + k * E
            pltpu.async_copy(cols_hbm.at[pl.ds(cbase0 + k * E, E)], IC[p], SI[p])
            pltpu.async_copy(vals16_hbm.at[pl.ds(base * 16, E * 16)], IV[p],
                             SI[p])
            pltpu.async_copy(rows_hbm.at[pl.ds(base, E)], IR[p], SI[p])

        def wait_idx(k, p):
            base = base0 + k * E
            pltpu.make_async_copy(cols_hbm.at[pl.ds(cbase0 + k * E, E)], IC[p],
                                  SI[p]).wait()
            pltpu.make_async_copy(vals16_hbm.at[pl.ds(base * 16, E * 16)],
                                  IV[p], SI[p]).wait()
            pltpu.make_async_copy(rows_hbm.at[pl.ds(base, E)], IR[p],
                                  SI[p]).wait()

        def step(k, p):
            wait_idx(k, p)
            issue_idx(k + 1, 1 - p)

            pltpu.async_copy(xsrc.at[IC[p]], rows_buf, gsem).wait()

            vb = IV[p]

            @plsc.parallel_loop(0, E)
            def _(e):
                vv = vb[pl.ds(e * 16, 16)]
                for j in range(DH // 16):
                    sl = pl.ds(j * 16, 16)
                    rows_buf[e, sl] = rows_buf[e, sl] * vv

            pltpu.sync_copy(rows_buf, acc.at[IR[p]], add=True)

        issue_idx(0, 0)

        def pair_body(i, carry):
            k0 = 2 * i
            step(k0, 0)
            step(k0 + 1, 1)
            return carry

        lax.fori_loop(0, NCHUNK // 2, pair_body, 0)
        wait_idx(NCHUNK, 0)
        plsc.subcore_barrier()
        pltpu.sync_copy(acc.at[pl.ds(s * RPT, RPT)],
                        dst.at[pl.ds(coff + s * RPT, RPT)])
        plsc.subcore_barrier()

    spmm(tr, tcl, tv16, xh, zr, m)
    spmm(sr, scl, sv16, m, xh, x1)
    spmm(tr, tcl, tv16, x1, zr, m)
    spmm(sr, scl, sv16, m, x1, x2)
    spmm(tr, tcl, tv16, x2, zr, m)
    spmm(sr, scl, sv16, m, x2, x3)

    def mean_chunk(k, carry):
        cid = s + k * NS

        @pl.when(cid < NCH)
        def _():
            r0 = cid * CH
            pltpu.sync_copy(xh.at[pl.ds(coff + r0, CH)], rows_buf.at[pl.ds(0, CH)])
            pltpu.sync_copy(x1.at[pl.ds(coff + r0, CH)], rows_buf.at[pl.ds(CH, CH)])
            pltpu.sync_copy(x2.at[pl.ds(coff + r0, CH)], rows_buf.at[pl.ds(2 * CH, CH)])
            pltpu.sync_copy(x3.at[pl.ds(coff + r0, CH)], rows_buf.at[pl.ds(3 * CH, CH)])

            def mean_body(r, cy):
                for j in range(DH // 16):
                    sl = pl.ds(j * 16, 16)
                    rows_buf[r, sl] = (rows_buf[r, sl] + rows_buf[CH + r, sl]
                                       + rows_buf[2 * CH + r, sl]
                                       + rows_buf[3 * CH + r, sl]) * 0.25
                return cy

            lax.fori_loop(0, CH, mean_body, 0)
            pltpu.sync_copy(rows_buf.at[pl.ds(0, CH)],
                            out.at[pl.ds(r0, CH), pl.ds(c * DH, DH)])

        return carry

    lax.fori_loop(0, KMAX, mean_chunk, 0)


_mesh = plsc.VectorSubcoreMesh(core_axis_name="c", subcore_axis_name="s")

_call = pl.kernel(
    _body,
    out_type=jax.ShapeDtypeStruct((N, D), jnp.float32),
    mesh=_mesh,
    scratch_types=[
        pltpu.HBM((2 * NPAD, DH), jnp.float32),
        pltpu.HBM((2 * NPAD, DH), jnp.float32),
        pltpu.HBM((2 * NPAD, DH), jnp.float32),
        pltpu.HBM((2 * NPAD, DH), jnp.float32),
        pltpu.VMEM((E,), jnp.int32),
        pltpu.VMEM((E,), jnp.int32),
        pltpu.VMEM((E * 16,), jnp.float32),
        pltpu.VMEM((E * 16,), jnp.float32),
        pltpu.VMEM((E,), jnp.int32),
        pltpu.VMEM((E,), jnp.int32),
        pltpu.VMEM((E, DH), jnp.float32),
        pltpu.VMEM_SHARED((NPAD, DH), jnp.float32),
        pltpu.SemaphoreType.DMA,
        pltpu.SemaphoreType.DMA,
        pltpu.SemaphoreType.DMA,
    ],
)


def kernel(pois_embs, src_indices, src_values, tar_indices, tar_values):
    xh = jnp.zeros((2 * NPAD, DH), jnp.float32)
    xh = xh.at[:N].set(pois_embs[:, :DH]).at[NPAD:NPAD + N].set(pois_embs[:, DH:])

    def pad_edges(indices, values):
        rows = jnp.zeros((CSTRIDE,), jnp.int32).at[:NNZ].set(
            indices[0].astype(jnp.int32))
        cols = jnp.zeros((CSTRIDE,), jnp.int32).at[:NNZ].set(
            indices[1].astype(jnp.int32))
        vals = jnp.zeros((CSTRIDE,), jnp.float32).at[:NNZ].set(values)
        cols2 = jnp.concatenate([cols, cols + NPAD])
        vals16 = jnp.repeat(vals, 16)
        return rows, cols2, vals16

    tr, tcl, tv16 = pad_edges(tar_indices, tar_values)
    sr, scl, sv16 = pad_edges(src_indices, src_values)
    zr = jnp.zeros((2 * NPAD, DH), jnp.float32)
    return _call(xh, tr, tcl, tv16, sr, scl, sv16, zr)


# speedup vs baseline: 1.4355x; 1.0288x over previous
import jax
import jax.numpy as jnp
from jax import lax
from jax.experimental import pallas as pl
from jax.experimental.pallas import tpu as pltpu
from jax.experimental.pallas import tpu_sc as plsc

N = 10000
D = 256
DH = 128
NNZ = 160000

NC = 2
NS = 16
NPAD = 10112
NNZP = 163840
EPT = NNZP // NS
E = 160
NCHUNK = EPT // E
RPT = NPAD // NS
CH = 40
NCH = N // CH
KMAX = (NCH + NS - 1) // NS
CSTRIDE = NNZP + 2 * E


def _body(xh, tr, tcl, tv16, sr, scl, sv16, zr, out,
          m, x1, x2, x3,
          b0, b1, ic0, ic1, iv0, iv1, ir0, ir1, acc,
          gsem, icvsem, irsem, ssem0, ssem1):
    c = lax.axis_index("c")
    s = lax.axis_index("s")
    coff = c * NPAD

    B = [b0, b1]
    IC = [ic0, ic1]
    IV = [iv0, iv1]
    IR = [ir0, ir1]
    SS = [ssem0, ssem1]

    base0 = s * EPT
    cbase0 = c * CSTRIDE + base0

    def spmm(rows_hbm, cols_hbm, vals16_hbm, xsrc, init, dst):
        pltpu.sync_copy(init.at[pl.ds(coff + s * RPT, RPT)],
                        acc.at[pl.ds(s * RPT, RPT)])
        plsc.subcore_barrier()

        def issue_icv(k, p):
            pltpu.async_copy(cols_hbm.at[pl.ds(cbase0 + k * E, E)], IC[p],
                             icvsem)
            pltpu.async_copy(
                vals16_hbm.at[pl.ds((base0 + k * E) * 16, E * 16)], IV[p],
                icvsem)

        def wait_icv(k, p):
            pltpu.make_async_copy(cols_hbm.at[pl.ds(cbase0 + k * E, E)], IC[p],
                                  icvsem).wait()
            pltpu.make_async_copy(
                vals16_hbm.at[pl.ds((base0 + k * E) * 16, E * 16)], IV[p],
                icvsem).wait()

        def issue_ir(k, p):
            pltpu.async_copy(rows_hbm.at[pl.ds(base0 + k * E, E)], IR[p],
                             irsem)

        def wait_ir(k, p):
            pltpu.make_async_copy(rows_hbm.at[pl.ds(base0 + k * E, E)], IR[p],
                                  irsem).wait()

        def issue_gather(p):
            pltpu.async_copy(xsrc.at[IC[p]], B[p], gsem)

        def wait_gather(p):
            pltpu.make_async_copy(xsrc.at[IC[p]], B[p], gsem).wait()

        def issue_scatter(p):
            pltpu.async_copy(B[p], acc.at[IR[p]], SS[p], add=True)

        def wait_scatter(p):
            pltpu.make_async_copy(B[p], acc.at[IR[p]], SS[p]).wait()

        def step(k, p, first=False):
            wait_gather(p)

            rb, vb = B[p], IV[p]

            @plsc.parallel_loop(0, E)
            def _(e):
                vv = vb[pl.ds(e * 16, 16)]
                for j in range(DH // 16):
                    sl = pl.ds(j * 16, 16)
                    rb[e, sl] = rb[e, sl] * vv

            wait_ir(k, p)
            issue_scatter(p)
            if not first:
                wait_scatter(1 - p)
            issue_ir(k + 1, 1 - p)
            wait_icv(k + 1, 1 - p)
            issue_gather(1 - p)
            issue_icv(k + 2, p)

        issue_icv(0, 0)
        issue_ir(0, 0)
        issue_icv(1, 1)
        wait_icv(0, 0)
        issue_gather(0)

        step(0, 0, first=True)

        def pair_body(i, carry):
            step(2 * i + 1, 1)
            step(2 * i + 2, 0)
            return carry

        lax.fori_loop(0, (NCHUNK - 2) // 2, pair_body, 0)
        step(NCHUNK - 1, 1)

        wait_scatter(1)
        wait_gather(0)
        wait_ir(NCHUNK, 0)
        wait_icv(NCHUNK + 1, 1)
        plsc.subcore_barrier()
        pltpu.sync_copy(acc.at[pl.ds(s * RPT, RPT)],
                        dst.at[pl.ds(coff + s * RPT, RPT)])
        plsc.subcore_barrier()

    spmm(tr, tcl, tv16, xh, zr, m)
    spmm(sr, scl, sv16, m, xh, x1)
    spmm(tr, tcl, tv16, x1, zr, m)
    spmm(sr, scl, sv16, m, x1, x2)
    spmm(tr, tcl, tv16, x2, zr, m)
    spmm(sr, scl, sv16, m, x2, x3)

    def mean_chunk(k, carry):
        cid = s + k * NS

        @pl.when(cid < NCH)
        def _():
            r0 = cid * CH
            pltpu.sync_copy(xh.at[pl.ds(coff + r0, CH)], b0.at[pl.ds(0, CH)])
            pltpu.sync_copy(x1.at[pl.ds(coff + r0, CH)], b0.at[pl.ds(CH, CH)])
            pltpu.sync_copy(x2.at[pl.ds(coff + r0, CH)], b1.at[pl.ds(0, CH)])
            pltpu.sync_copy(x3.at[pl.ds(coff + r0, CH)], b1.at[pl.ds(CH, CH)])

            def mean_body(r, cy):
                for j in range(DH // 16):
                    sl = pl.ds(j * 16, 16)
                    b0[r, sl] = (b0[r, sl] + b0[CH + r, sl]
                                 + b1[r, sl] + b1[CH + r, sl]) * 0.25
                return cy

            lax.fori_loop(0, CH, mean_body, 0)
            pltpu.sync_copy(b0.at[pl.ds(0, CH)],
                            out.at[pl.ds(r0, CH), pl.ds(c * DH, DH)])

        return carry

    lax.fori_loop(0, KMAX, mean_chunk, 0)


_mesh = plsc.VectorSubcoreMesh(core_axis_name="c", subcore_axis_name="s")

_call = pl.kernel(
    _body,
    out_type=jax.ShapeDtypeStruct((N, D), jnp.float32),
    mesh=_mesh,
    scratch_types=[
        pltpu.HBM((2 * NPAD, DH), jnp.float32),
        pltpu.HBM((2 * NPAD, DH), jnp.float32),
        pltpu.HBM((2 * NPAD, DH), jnp.float32),
        pltpu.HBM((2 * NPAD, DH), jnp.float32),
        pltpu.VMEM((E, DH), jnp.float32),
        pltpu.VMEM((E, DH), jnp.float32),
        pltpu.VMEM((E,), jnp.int32),
        pltpu.VMEM((E,), jnp.int32),
        pltpu.VMEM((E * 16,), jnp.float32),
        pltpu.VMEM((E * 16,), jnp.float32),
        pltpu.VMEM((E,), jnp.int32),
        pltpu.VMEM((E,), jnp.int32),
        pltpu.VMEM_SHARED((NPAD, DH), jnp.float32),
        pltpu.SemaphoreType.DMA,
        pltpu.SemaphoreType.DMA,
        pltpu.SemaphoreType.DMA,
        pltpu.SemaphoreType.DMA,
        pltpu.SemaphoreType.DMA,
    ],
)


def kernel(pois_embs, src_indices, src_values, tar_indices, tar_values):
    xh = jnp.zeros((2 * NPAD, DH), jnp.float32)
    xh = xh.at[:N].set(pois_embs[:, :DH]).at[NPAD:NPAD + N].set(pois_embs[:, DH:])

    def pad_edges(indices, values):
        rows = jnp.zeros((CSTRIDE,), jnp.int32).at[:NNZ].set(
            indices[0].astype(jnp.int32))
        cols = jnp.zeros((CSTRIDE,), jnp.int32).at[:NNZ].set(
            indices[1].astype(jnp.int32))
        vals = jnp.zeros((CSTRIDE,), jnp.float32).at[:NNZ].set(values)
        cols2 = jnp.concatenate([cols, cols + NPAD])
        vals16 = jnp.repeat(vals, 16)
        return rows, cols2, vals16

    tr, tcl, tv16 = pad_edges(tar_indices, tar_values)
    sr, scl, sv16 = pad_edges(src_indices, src_values)
    zr = jnp.zeros((2 * NPAD, DH), jnp.float32)
    return _call(xh, tr, tcl, tv16, sr, scl, sv16, zr)
